# baseline (speedup 1.0000x reference)
import functools

import jax
import jax.numpy as jnp
import numpy as np
from jax import lax
from jax.experimental import pallas as pl
from jax.experimental.pallas import tpu as pltpu
from jax.experimental.pallas import tpu_sc as plsc

N_NODES = 10000
N_EDGES = 320000
NC = 2
NS = 16
EPS_SUB = N_EDGES // NS
CK = 100
NCH = EPS_SUB // CK
GCH = 40
NG = NCH // GCH
SUB_ROWS = 624
TAIL_OFF = NS * SUB_ROWS
TAIL_ROWS = N_NODES - TAIL_OFF


@functools.lru_cache(maxsize=None)
def _make_segsum(d_feat):
    mesh = plsc.VectorSubcoreMesh(core_axis_name="c", subcore_axis_name="s",
                                  num_cores=NC, num_subcores=NS)
    cparams = (None if d_feat == 128
               else pltpu.CompilerParams(use_tc_tiling_on_sc=False))

    @functools.partial(
        pl.kernel,
        mesh=mesh,
        compiler_params=cparams,
        out_type=jax.ShapeDtypeStruct((NC, N_NODES, d_feat), jnp.float32),
        scratch_types=[
            pltpu.VMEM((GCH, CK), jnp.int32),
            pltpu.VMEM((GCH, CK), jnp.int32),
            pltpu.VMEM((CK, d_feat), jnp.float32),
            pltpu.VMEM((CK, d_feat), jnp.float32),
            pltpu.VMEM_SHARED((N_NODES, d_feat), jnp.float32),
            pltpu.SemaphoreType.DMA,
            pltpu.SemaphoreType.DMA,
        ],
    )
    def segsum(x1_hbm, x2_hbm, srcr_hbm, dstr_hbm, zeros_hbm, out_hbm,
               src_v, dst_v, rows0, rows1, acc_sh, gsem0, gsem1):
        c = lax.axis_index("c")
        s = lax.axis_index("s")
        off = pl.multiple_of(s * SUB_ROWS, 8)
        pltpu.sync_copy(zeros_hbm.at[pl.ds(off, SUB_ROWS)],
                        acc_sh.at[pl.ds(off, SUB_ROWS)])

        @pl.when(s == 0)
        def _():
            pltpu.sync_copy(zeros_hbm.at[pl.ds(TAIL_OFF, TAIL_ROWS)],
                            acc_sh.at[pl.ds(TAIL_OFF, TAIL_ROWS)])
        wid = c * NS + s
        plsc.subcore_barrier()

        def run(x_hbm):
            for grp in range(NG):
                pltpu.sync_copy(
                    srcr_hbm.at[wid].at[pl.ds(grp * GCH, GCH)], src_v)
                pltpu.sync_copy(
                    dstr_hbm.at[wid].at[pl.ds(grp * GCH, GCH)], dst_v)
                pltpu.async_copy(x_hbm.at[src_v.at[0]], rows0, gsem0)

                def body2(i, carry):
                    j0 = 2 * i
                    j1 = j0 + 1
                    pltpu.make_async_copy(
                        x_hbm.at[src_v.at[j0]], rows0, gsem0).wait()
                    pltpu.async_copy(x_hbm.at[src_v.at[j1]], rows1, gsem1)
                    pltpu.sync_copy(rows0, acc_sh.at[dst_v.at[j0]], add=True)
                    pltpu.make_async_copy(
                        x_hbm.at[src_v.at[j1]], rows1, gsem1).wait()

                    @pl.when(j1 + 1 < GCH)
                    def _():
                        pltpu.async_copy(
                            x_hbm.at[src_v.at[j1 + 1]], rows0, gsem0)

                    pltpu.sync_copy(rows1, acc_sh.at[dst_v.at[j1]], add=True)
                    return carry

                lax.fori_loop(0, GCH // 2, body2, 0, unroll=False)

        @pl.when(c == 0)
        def _():
            run(x1_hbm)

        @pl.when(c == 1)
        def _():
            run(x2_hbm)

        plsc.subcore_barrier()
        pltpu.sync_copy(acc_sh.at[pl.ds(off, SUB_ROWS)],
                        out_hbm.at[c].at[pl.ds(off, SUB_ROWS)])

        @pl.when(s == 0)
        def _():
            pltpu.sync_copy(acc_sh.at[pl.ds(TAIL_OFF, TAIL_ROWS)],
                            out_hbm.at[c].at[pl.ds(TAIL_OFF, TAIL_ROWS)])

    return segsum


def _segsum_both(x1, x2, srcr, dstr, zeros):
    return _make_segsum(x1.shape[1])(x1, x2, srcr, dstr, zeros)


_BLK = 1000


def _dense_layer_body(use_bn, x_r, agg_r, eps_r, w1_r, b1_r, w2_r, b2_r,
                      g_r, bta_r, wi_r, bi_r, c_r, pool_r):
    h = (1.0 + eps_r[0, 0]) * x_r[...] + agg_r[0]
    h = jnp.maximum(
        jnp.dot(h, w1_r[...], preferred_element_type=jnp.float32) + b1_r[...],
        0.0)
    h = jnp.dot(h, w2_r[...], preferred_element_type=jnp.float32) + b2_r[...]
    if use_bn:
        h = h * (g_r[...] / np.sqrt(1.0 + 1e-5)) + bta_r[...]
    else:
        mu = jnp.mean(h, axis=1, keepdims=True)
        var = jnp.mean((h - mu) ** 2, axis=1, keepdims=True)
        h = (h - mu) * lax.rsqrt(var + 1e-5) * g_r[...] + bta_r[...]
    c = jnp.maximum(h, 0.0)
    c_r[...] = c
    inner = jnp.maximum(
        jnp.dot(c, wi_r[...], preferred_element_type=jnp.float32) + bi_r[...],
        0.0)
    psum = jnp.sum(inner, axis=0, keepdims=True)

    @pl.when(pl.program_id(0) == 0)
    def _():
        pool_r[...] = jnp.zeros_like(pool_r)

    pool_r[...] += psum


def _dense_layer(x, agg, g_idx, gin_p, inner_p, use_bn):
    n, din = x.shape
    w1 = gin_p['W1']
    f = w1.shape[1]
    grid = (n // _BLK,)
    full = lambda i: (0, 0)
    body = functools.partial(_dense_layer_body, use_bn)
    c, pool = pl.pallas_call(
        body,
        grid=grid,
        in_specs=[
            pl.BlockSpec((_BLK, din), lambda i: (i, 0)),
            pl.BlockSpec((1, _BLK, din), lambda i: (g_idx, i, 0)),
            pl.BlockSpec((1, 1), full),
            pl.BlockSpec((din, f), full),
            pl.BlockSpec((1, f), full),
            pl.BlockSpec((f, f), full),
            pl.BlockSpec((1, f), full),
            pl.BlockSpec((1, f), full),
            pl.BlockSpec((1, f), full),
            pl.BlockSpec((f, f), full),
            pl.BlockSpec((1, f), full),
        ],
        out_specs=[
            pl.BlockSpec((_BLK, f), lambda i: (i, 0)),
            pl.BlockSpec((1, f), full),
        ],
        out_shape=[
            jax.ShapeDtypeStruct((n, f), jnp.float32),
            jax.ShapeDtypeStruct((1, f), jnp.float32),
        ],
    )(x, agg,
      gin_p['eps'].reshape(1, 1),
      w1, gin_p['b1'].reshape(1, f),
      gin_p['W2'], gin_p['b2'].reshape(1, f),
      gin_p['g'].reshape(1, f), gin_p['bta'].reshape(1, f),
      inner_p['W'], inner_p['b'].reshape(1, f))
    return c, pool


def _head_body(p1_0, p1_1, p1_2, p2_0, p2_1, p2_2,
               wo0, bo0, wo1, bo1, wo2, bo2,
               wc1, bc1, wc2, bc2,
               w3, vt, b3, ws1, bs1, ws2, bs2,
               wt1, bt1, wt2, bt2, alpha, beta, out_r):
    pools1 = [p1_0, p1_1, p1_2]
    pools2 = [p2_0, p2_1, p2_2]
    wos = [wo0, wo1, wo2]
    bos = [bo0, bo1, bo2]
    diffs = []
    hi = hj = None
    for i in range(3):
        d1 = jnp.maximum(
            jnp.dot(pools1[i][...], wos[i][...],
                    preferred_element_type=jnp.float32) + bos[i][...], 0.0)
        d2 = jnp.maximum(
            jnp.dot(pools2[i][...], wos[i][...],
                    preferred_element_type=jnp.float32) + bos[i][...], 0.0)
        diffs.append(jnp.exp(-(d1 - d2) ** 2))
        hi, hj = d1, d2
    diff_rep = jnp.concatenate(diffs, axis=1)
    h = jnp.maximum(
        jnp.dot(diff_rep, wc1[...], preferred_element_type=jnp.float32)
        + bc1[...], 0.0)
    score_rep = jnp.tanh(
        jnp.dot(h, wc2[...], preferred_element_type=jnp.float32) + bc2[...])
    t1 = []
    for k in range(16):
        v = jnp.dot(hi, w3[k], preferred_element_type=jnp.float32)
        t1.append(jnp.sum(v * hj, axis=1, keepdims=True))
    term_1 = jnp.concatenate(t1, axis=1)
    hij = jnp.concatenate([hi, hj], axis=1)
    term_2 = jnp.dot(hij, vt[...], preferred_element_type=jnp.float32)
    sim_rep = jnp.maximum(term_1 + term_2 + b3[...], 0.0)
    s = jnp.dot(
        jnp.maximum(jnp.dot(score_rep, ws1[...],
                            preferred_element_type=jnp.float32) + bs1[...],
                    0.0),
        ws2[...], preferred_element_type=jnp.float32) + bs2[...]
    score = jax.nn.sigmoid(s)
    t = jnp.dot(
        jnp.maximum(jnp.dot(sim_rep, wt1[...],
                            preferred_element_type=jnp.float32) + bt1[...],
                    0.0),
        wt2[...], preferred_element_type=jnp.float32) + bt2[...]
    sim_score = jax.nn.sigmoid(t)
    out_r[...] = alpha[...] * score + beta[...] * sim_score


def _head(pools1, pools2, p):
    f = 64
    args = [
        pools1[0], pools1[1], pools1[2], pools2[0], pools2[1], pools2[2],
        p['outer'][0]['W'], p['outer'][0]['b'].reshape(1, f),
        p['outer'][1]['W'], p['outer'][1]['b'].reshape(1, f),
        p['outer'][2]['W'], p['outer'][2]['b'].reshape(1, f),
        p['Wc1'], p['bc1'].reshape(1, -1),
        p['Wc2'], p['bc2'].reshape(1, -1),
        p['W3'], p['V'].T, p['b3'],
        p['Ws1'], p['bs1'].reshape(1, -1),
        p['Ws2'], p['bs2'].reshape(1, -1),
        p['Wt1'], p['bt1'].reshape(1, -1),
        p['Wt2'], p['bt2'].reshape(1, -1),
        p['alpha'].reshape(1, 1), p['beta'].reshape(1, 1),
    ]
    out = pl.pallas_call(
        _head_body,
        out_shape=jax.ShapeDtypeStruct((1, 1), jnp.float32),
    )(*args)
    return out.reshape(-1)


def kernel(x1, edge_index_1, x2, edge_index_2, params):
    p = params
    srcr = jnp.concatenate([edge_index_1[0], edge_index_2[0]]
                           ).reshape(NC * NS, NCH, CK)
    dstr = jnp.concatenate([edge_index_1[1], edge_index_2[1]]
                           ).reshape(NC * NS, NCH, CK)
    zeros = {d: jnp.zeros((N_NODES, d), jnp.float32) for d in (128, 64)}

    c1, c2 = x1, x2
    pools1, pools2 = [], []
    for i in range(3):
        agg = _segsum_both(c1, c2, srcr, dstr, zeros[c1.shape[1]])
        c1, pool1 = _dense_layer(c1, agg, 0, p['gin'][i], p['inner'][i],
                                 i == 0)
        c2, pool2 = _dense_layer(c2, agg, 1, p['gin'][i], p['inner'][i],
                                 i == 0)
        pools1.append(pool1)
        pools2.append(pool2)
    return _head(pools1, pools2, p)


# speedup vs baseline: 1.1154x; 1.1154x over previous
import functools

import jax
import jax.numpy as jnp
import numpy as np
from jax import lax
from jax.experimental import pallas as pl
from jax.experimental.pallas import tpu as pltpu
from jax.experimental.pallas import tpu_sc as plsc

N_NODES = 10000
N_EDGES = 320000
NC = 2
NS = 16
EPS_SUB = N_EDGES // NS
CK = 100
NCH = EPS_SUB // CK
GCH = 40
NG = NCH // GCH
SUB_ROWS = 624
TAIL_OFF = NS * SUB_ROWS
TAIL_ROWS = N_NODES - TAIL_OFF


@functools.lru_cache(maxsize=None)
def _make_segsum(d_feat):
    mesh = plsc.VectorSubcoreMesh(core_axis_name="c", subcore_axis_name="s",
                                  num_cores=NC, num_subcores=NS)
    cparams = (None if d_feat == 128
               else pltpu.CompilerParams(use_tc_tiling_on_sc=False))
    stage_x = d_feat == 64
    scratch = [
        pltpu.VMEM((GCH, CK), jnp.int32),
        pltpu.VMEM((GCH, CK), jnp.int32),
        pltpu.VMEM((CK, d_feat), jnp.float32),
        pltpu.VMEM((CK, d_feat), jnp.float32),
        pltpu.VMEM_SHARED((N_NODES, d_feat), jnp.float32),
        pltpu.SemaphoreType.DMA,
        pltpu.SemaphoreType.DMA,
    ]
    if stage_x:
        scratch.append(pltpu.VMEM_SHARED((N_NODES, d_feat), jnp.float32))

    @functools.partial(
        pl.kernel,
        mesh=mesh,
        compiler_params=cparams,
        out_type=jax.ShapeDtypeStruct((NC, N_NODES, d_feat), jnp.float32),
        scratch_types=scratch,
    )
    def segsum(x1_hbm, x2_hbm, srcr_hbm, dstr_hbm, zeros_hbm, out_hbm,
               src_v, dst_v, rows0, rows1, acc_sh, gsem0, gsem1,
               *maybe_x_sh):
        c = lax.axis_index("c")
        s = lax.axis_index("s")
        off = pl.multiple_of(s * SUB_ROWS, 8)
        pltpu.sync_copy(zeros_hbm.at[pl.ds(off, SUB_ROWS)],
                        acc_sh.at[pl.ds(off, SUB_ROWS)])

        @pl.when(s == 0)
        def _():
            pltpu.sync_copy(zeros_hbm.at[pl.ds(TAIL_OFF, TAIL_ROWS)],
                            acc_sh.at[pl.ds(TAIL_OFF, TAIL_ROWS)])
        wid = c * NS + s
        plsc.subcore_barrier()

        def run(x_hbm):
            if stage_x:
                x_sh = maybe_x_sh[0]
                pltpu.sync_copy(x_hbm.at[pl.ds(off, SUB_ROWS)],
                                x_sh.at[pl.ds(off, SUB_ROWS)])

                @pl.when(s == 0)
                def _():
                    pltpu.sync_copy(x_hbm.at[pl.ds(TAIL_OFF, TAIL_ROWS)],
                                    x_sh.at[pl.ds(TAIL_OFF, TAIL_ROWS)])

                plsc.subcore_barrier()
                x_src = x_sh
            else:
                x_src = x_hbm
            for grp in range(NG):
                pltpu.sync_copy(
                    srcr_hbm.at[wid].at[pl.ds(grp * GCH, GCH)], src_v)
                pltpu.sync_copy(
                    dstr_hbm.at[wid].at[pl.ds(grp * GCH, GCH)], dst_v)
                pltpu.async_copy(x_src.at[src_v.at[0]], rows0, gsem0)

                def body2(i, carry):
                    j0 = 2 * i
                    j1 = j0 + 1
                    pltpu.make_async_copy(
                        x_src.at[src_v.at[j0]], rows0, gsem0).wait()
                    pltpu.async_copy(x_src.at[src_v.at[j1]], rows1, gsem1)
                    pltpu.sync_copy(rows0, acc_sh.at[dst_v.at[j0]], add=True)
                    pltpu.make_async_copy(
                        x_src.at[src_v.at[j1]], rows1, gsem1).wait()

                    @pl.when(j1 + 1 < GCH)
                    def _():
                        pltpu.async_copy(
                            x_src.at[src_v.at[j1 + 1]], rows0, gsem0)

                    pltpu.sync_copy(rows1, acc_sh.at[dst_v.at[j1]], add=True)
                    return carry

                lax.fori_loop(0, GCH // 2, body2, 0, unroll=False)

        @pl.when(c == 0)
        def _():
            run(x1_hbm)

        @pl.when(c == 1)
        def _():
            run(x2_hbm)

        plsc.subcore_barrier()
        pltpu.sync_copy(acc_sh.at[pl.ds(off, SUB_ROWS)],
                        out_hbm.at[c].at[pl.ds(off, SUB_ROWS)])

        @pl.when(s == 0)
        def _():
            pltpu.sync_copy(acc_sh.at[pl.ds(TAIL_OFF, TAIL_ROWS)],
                            out_hbm.at[c].at[pl.ds(TAIL_OFF, TAIL_ROWS)])

    return segsum


def _segsum_both(x1, x2, srcr, dstr, zeros):
    return _make_segsum(x1.shape[1])(x1, x2, srcr, dstr, zeros)


_BLK = 1000


def _dense_layer_body(use_bn, x_r, agg_r, eps_r, w1_r, b1_r, w2_r, b2_r,
                      g_r, bta_r, wi_r, bi_r, c_r, pool_r):
    h = (1.0 + eps_r[0, 0]) * x_r[...] + agg_r[0]
    h = jnp.maximum(
        jnp.dot(h, w1_r[...], preferred_element_type=jnp.float32) + b1_r[...],
        0.0)
    h = jnp.dot(h, w2_r[...], preferred_element_type=jnp.float32) + b2_r[...]
    if use_bn:
        h = h * (g_r[...] / np.sqrt(1.0 + 1e-5)) + bta_r[...]
    else:
        mu = jnp.mean(h, axis=1, keepdims=True)
        var = jnp.mean((h - mu) ** 2, axis=1, keepdims=True)
        h = (h - mu) * lax.rsqrt(var + 1e-5) * g_r[...] + bta_r[...]
    c = jnp.maximum(h, 0.0)
    c_r[...] = c
    inner = jnp.maximum(
        jnp.dot(c, wi_r[...], preferred_element_type=jnp.float32) + bi_r[...],
        0.0)
    psum = jnp.sum(inner, axis=0, keepdims=True)

    @pl.when(pl.program_id(0) == 0)
    def _():
        pool_r[...] = jnp.zeros_like(pool_r)

    pool_r[...] += psum


def _dense_layer(x, agg, g_idx, gin_p, inner_p, use_bn):
    n, din = x.shape
    w1 = gin_p['W1']
    f = w1.shape[1]
    grid = (n // _BLK,)
    full = lambda i: (0, 0)
    body = functools.partial(_dense_layer_body, use_bn)
    c, pool = pl.pallas_call(
        body,
        grid=grid,
        in_specs=[
            pl.BlockSpec((_BLK, din), lambda i: (i, 0)),
            pl.BlockSpec((1, _BLK, din), lambda i: (g_idx, i, 0)),
            pl.BlockSpec((1, 1), full),
            pl.BlockSpec((din, f), full),
            pl.BlockSpec((1, f), full),
            pl.BlockSpec((f, f), full),
            pl.BlockSpec((1, f), full),
            pl.BlockSpec((1, f), full),
            pl.BlockSpec((1, f), full),
            pl.BlockSpec((f, f), full),
            pl.BlockSpec((1, f), full),
        ],
        out_specs=[
            pl.BlockSpec((_BLK, f), lambda i: (i, 0)),
            pl.BlockSpec((1, f), full),
        ],
        out_shape=[
            jax.ShapeDtypeStruct((n, f), jnp.float32),
            jax.ShapeDtypeStruct((1, f), jnp.float32),
        ],
    )(x, agg,
      gin_p['eps'].reshape(1, 1),
      w1, gin_p['b1'].reshape(1, f),
      gin_p['W2'], gin_p['b2'].reshape(1, f),
      gin_p['g'].reshape(1, f), gin_p['bta'].reshape(1, f),
      inner_p['W'], inner_p['b'].reshape(1, f))
    return c, pool


def _head_body(p1_0, p1_1, p1_2, p2_0, p2_1, p2_2,
               wo0, bo0, wo1, bo1, wo2, bo2,
               wc1, bc1, wc2, bc2,
               w3, vt, b3, ws1, bs1, ws2, bs2,
               wt1, bt1, wt2, bt2, alpha, beta, out_r):
    pools1 = [p1_0, p1_1, p1_2]
    pools2 = [p2_0, p2_1, p2_2]
    wos = [wo0, wo1, wo2]
    bos = [bo0, bo1, bo2]
    diffs = []
    hi = hj = None
    for i in range(3):
        d1 = jnp.maximum(
            jnp.dot(pools1[i][...], wos[i][...],
                    preferred_element_type=jnp.float32) + bos[i][...], 0.0)
        d2 = jnp.maximum(
            jnp.dot(pools2[i][...], wos[i][...],
                    preferred_element_type=jnp.float32) + bos[i][...], 0.0)
        diffs.append(jnp.exp(-(d1 - d2) ** 2))
        hi, hj = d1, d2
    diff_rep = jnp.concatenate(diffs, axis=1)
    h = jnp.maximum(
        jnp.dot(diff_rep, wc1[...], preferred_element_type=jnp.float32)
        + bc1[...], 0.0)
    score_rep = jnp.tanh(
        jnp.dot(h, wc2[...], preferred_element_type=jnp.float32) + bc2[...])
    t1 = []
    for k in range(16):
        v = jnp.dot(hi, w3[k], preferred_element_type=jnp.float32)
        t1.append(jnp.sum(v * hj, axis=1, keepdims=True))
    term_1 = jnp.concatenate(t1, axis=1)
    hij = jnp.concatenate([hi, hj], axis=1)
    term_2 = jnp.dot(hij, vt[...], preferred_element_type=jnp.float32)
    sim_rep = jnp.maximum(term_1 + term_2 + b3[...], 0.0)
    s = jnp.dot(
        jnp.maximum(jnp.dot(score_rep, ws1[...],
                            preferred_element_type=jnp.float32) + bs1[...],
                    0.0),
        ws2[...], preferred_element_type=jnp.float32) + bs2[...]
    score = jax.nn.sigmoid(s)
    t = jnp.dot(
        jnp.maximum(jnp.dot(sim_rep, wt1[...],
                            preferred_element_type=jnp.float32) + bt1[...],
                    0.0),
        wt2[...], preferred_element_type=jnp.float32) + bt2[...]
    sim_score = jax.nn.sigmoid(t)
    out_r[...] = alpha[...] * score + beta[...] * sim_score


def _head(pools1, pools2, p):
    f = 64
    args = [
        pools1[0], pools1[1], pools1[2], pools2[0], pools2[1], pools2[2],
        p['outer'][0]['W'], p['outer'][0]['b'].reshape(1, f),
        p['outer'][1]['W'], p['outer'][1]['b'].reshape(1, f),
        p['outer'][2]['W'], p['outer'][2]['b'].reshape(1, f),
        p['Wc1'], p['bc1'].reshape(1, -1),
        p['Wc2'], p['bc2'].reshape(1, -1),
        p['W3'], p['V'].T, p['b3'],
        p['Ws1'], p['bs1'].reshape(1, -1),
        p['Ws2'], p['bs2'].reshape(1, -1),
        p['Wt1'], p['bt1'].reshape(1, -1),
        p['Wt2'], p['bt2'].reshape(1, -1),
        p['alpha'].reshape(1, 1), p['beta'].reshape(1, 1),
    ]
    out = pl.pallas_call(
        _head_body,
        out_shape=jax.ShapeDtypeStruct((1, 1), jnp.float32),
    )(*args)
    return out.reshape(-1)


def kernel(x1, edge_index_1, x2, edge_index_2, params):
    p = params
    srcr = jnp.concatenate([edge_index_1[0], edge_index_2[0]]
                           ).reshape(NC * NS, NCH, CK)
    dstr = jnp.concatenate([edge_index_1[1], edge_index_2[1]]
                           ).reshape(NC * NS, NCH, CK)
    zeros = {d: jnp.zeros((N_NODES, d), jnp.float32) for d in (128, 64)}

    c1, c2 = x1, x2
    pools1, pools2 = [], []
    for i in range(3):
        agg = _segsum_both(c1, c2, srcr, dstr, zeros[c1.shape[1]])
        c1, pool1 = _dense_layer(c1, agg, 0, p['gin'][i], p['inner'][i],
                                 i == 0)
        c2, pool2 = _dense_layer(c2, agg, 1, p['gin'][i], p['inner'][i],
                                 i == 0)
        pools1.append(pool1)
        pools2.append(pool2)
    return _head(pools1, pools2, p)


# speedup vs baseline: 1.1728x; 1.0515x over previous
import functools

import jax
import jax.numpy as jnp
import numpy as np
from jax import lax
from jax.experimental import pallas as pl
from jax.experimental.pallas import tpu as pltpu
from jax.experimental.pallas import tpu_sc as plsc

N_NODES = 10000
N_EDGES = 320000
NC = 2
NS = 16
EPS_SUB = N_EDGES // NS
GCH = 40
SUB_ROWS = 624
TAIL_OFF = NS * SUB_ROWS
TAIL_ROWS = N_NODES - TAIL_OFF


@functools.lru_cache(maxsize=None)
def _make_segsum(d_feat):
    mesh = plsc.VectorSubcoreMesh(core_axis_name="c", subcore_axis_name="s",
                                  num_cores=NC, num_subcores=NS)
    cparams = (None if d_feat == 128
               else pltpu.CompilerParams(use_tc_tiling_on_sc=False))
    ck = 50 if d_feat == 128 else 100
    nch = EPS_SUB // ck
    ng = nch // GCH
    scratch = (
        [pltpu.VMEM((GCH, ck), jnp.int32),
         pltpu.VMEM((GCH, ck), jnp.int32)]
        + [pltpu.VMEM((ck, d_feat), jnp.float32) for _ in range(4)]
        + [pltpu.VMEM_SHARED((N_NODES, d_feat), jnp.float32)]
        + [pltpu.SemaphoreType.DMA] * 8
    )

    @functools.partial(
        pl.kernel,
        mesh=mesh,
        compiler_params=cparams,
        out_type=jax.ShapeDtypeStruct((NC, N_NODES, d_feat), jnp.float32),
        scratch_types=scratch,
    )
    def segsum(x1_hbm, x2_hbm, srcr_hbm, dstr_hbm, zeros_hbm, out_hbm,
               src_v, dst_v, r0, r1, r2, r3, acc_sh,
               g0, g1, g2, g3, s0, s1, s2, s3):
        rows = [r0, r1, r2, r3]
        gsem = [g0, g1, g2, g3]
        ssem = [s0, s1, s2, s3]
        c = lax.axis_index("c")
        s = lax.axis_index("s")
        off = pl.multiple_of(s * SUB_ROWS, 8)
        pltpu.sync_copy(zeros_hbm.at[pl.ds(off, SUB_ROWS)],
                        acc_sh.at[pl.ds(off, SUB_ROWS)])

        @pl.when(s == 0)
        def _():
            pltpu.sync_copy(zeros_hbm.at[pl.ds(TAIL_OFF, TAIL_ROWS)],
                            acc_sh.at[pl.ds(TAIL_OFF, TAIL_ROWS)])
        wid = c * NS + s
        plsc.subcore_barrier()

        def run(x_hbm):
            for grp in range(ng):
                pltpu.sync_copy(
                    srcr_hbm.at[wid].at[pl.ds(grp * GCH, GCH)], src_v)
                pltpu.sync_copy(
                    dstr_hbm.at[wid].at[pl.ds(grp * GCH, GCH)], dst_v)
                pltpu.async_copy(x_hbm.at[src_v.at[0]], rows[0], gsem[0])
                pltpu.async_copy(x_hbm.at[src_v.at[1]], rows[1], gsem[1])

                def body4(q, carry):
                    for k in range(4):
                        j = 4 * q + k
                        kp = (k + 2) % 4
                        pltpu.make_async_copy(
                            x_hbm.at[src_v.at[j]], rows[k], gsem[k]).wait()
                        pltpu.async_copy(
                            rows[k], acc_sh.at[dst_v.at[j]], ssem[k],
                            add=True)

                        @pl.when(j >= 2)
                        def _():
                            pltpu.make_async_copy(
                                rows[kp], acc_sh.at[dst_v.at[j - 2]],
                                ssem[kp]).wait()

                        @pl.when(j + 2 < GCH)
                        def _():
                            pltpu.async_copy(
                                x_hbm.at[src_v.at[j + 2]], rows[kp],
                                gsem[kp])
                    return carry

                lax.fori_loop(0, GCH // 4, body4, 0, unroll=False)
                pltpu.make_async_copy(
                    rows[(GCH - 2) % 4], acc_sh.at[dst_v.at[GCH - 2]],
                    ssem[(GCH - 2) % 4]).wait()
                pltpu.make_async_copy(
                    rows[(GCH - 1) % 4], acc_sh.at[dst_v.at[GCH - 1]],
                    ssem[(GCH - 1) % 4]).wait()

        @pl.when(c == 0)
        def _():
            run(x1_hbm)

        @pl.when(c == 1)
        def _():
            run(x2_hbm)

        plsc.subcore_barrier()
        pltpu.sync_copy(acc_sh.at[pl.ds(off, SUB_ROWS)],
                        out_hbm.at[c].at[pl.ds(off, SUB_ROWS)])

        @pl.when(s == 0)
        def _():
            pltpu.sync_copy(acc_sh.at[pl.ds(TAIL_OFF, TAIL_ROWS)],
                            out_hbm.at[c].at[pl.ds(TAIL_OFF, TAIL_ROWS)])

    return segsum


def _segsum_both(x1, x2, srcr, dstr, zeros):
    d = x1.shape[1]
    ck = 50 if d == 128 else 100
    nch = EPS_SUB // ck
    return _make_segsum(d)(x1, x2,
                           srcr.reshape(NC * NS, nch, ck),
                           dstr.reshape(NC * NS, nch, ck), zeros)


_BLK = 1000


def _dense_layer_body(use_bn, x_r, agg_r, eps_r, w1_r, b1_r, w2_r, b2_r,
                      g_r, bta_r, wi_r, bi_r, c_r, pool_r):
    h = (1.0 + eps_r[0, 0]) * x_r[...] + agg_r[0]
    h = jnp.maximum(
        jnp.dot(h, w1_r[...], preferred_element_type=jnp.float32) + b1_r[...],
        0.0)
    h = jnp.dot(h, w2_r[...], preferred_element_type=jnp.float32) + b2_r[...]
    if use_bn:
        h = h * (g_r[...] / np.sqrt(1.0 + 1e-5)) + bta_r[...]
    else:
        mu = jnp.mean(h, axis=1, keepdims=True)
        var = jnp.mean((h - mu) ** 2, axis=1, keepdims=True)
        h = (h - mu) * lax.rsqrt(var + 1e-5) * g_r[...] + bta_r[...]
    c = jnp.maximum(h, 0.0)
    c_r[...] = c
    inner = jnp.maximum(
        jnp.dot(c, wi_r[...], preferred_element_type=jnp.float32) + bi_r[...],
        0.0)
    psum = jnp.sum(inner, axis=0, keepdims=True)

    @pl.when(pl.program_id(0) == 0)
    def _():
        pool_r[...] = jnp.zeros_like(pool_r)

    pool_r[...] += psum


def _dense_layer(x, agg, g_idx, gin_p, inner_p, use_bn):
    n, din = x.shape
    w1 = gin_p['W1']
    f = w1.shape[1]
    grid = (n // _BLK,)
    full = lambda i: (0, 0)
    body = functools.partial(_dense_layer_body, use_bn)
    c, pool = pl.pallas_call(
        body,
        grid=grid,
        in_specs=[
            pl.BlockSpec((_BLK, din), lambda i: (i, 0)),
            pl.BlockSpec((1, _BLK, din), lambda i: (g_idx, i, 0)),
            pl.BlockSpec((1, 1), full),
            pl.BlockSpec((din, f), full),
            pl.BlockSpec((1, f), full),
            pl.BlockSpec((f, f), full),
            pl.BlockSpec((1, f), full),
            pl.BlockSpec((1, f), full),
            pl.BlockSpec((1, f), full),
            pl.BlockSpec((f, f), full),
            pl.BlockSpec((1, f), full),
        ],
        out_specs=[
            pl.BlockSpec((_BLK, f), lambda i: (i, 0)),
            pl.BlockSpec((1, f), full),
        ],
        out_shape=[
            jax.ShapeDtypeStruct((n, f), jnp.float32),
            jax.ShapeDtypeStruct((1, f), jnp.float32),
        ],
    )(x, agg,
      gin_p['eps'].reshape(1, 1),
      w1, gin_p['b1'].reshape(1, f),
      gin_p['W2'], gin_p['b2'].reshape(1, f),
      gin_p['g'].reshape(1, f), gin_p['bta'].reshape(1, f),
      inner_p['W'], inner_p['b'].reshape(1, f))
    return c, pool


def _head_body(p1_0, p1_1, p1_2, p2_0, p2_1, p2_2,
               wo0, bo0, wo1, bo1, wo2, bo2,
               wc1, bc1, wc2, bc2,
               w3, vt, b3, ws1, bs1, ws2, bs2,
               wt1, bt1, wt2, bt2, alpha, beta, out_r):
    pools1 = [p1_0, p1_1, p1_2]
    pools2 = [p2_0, p2_1, p2_2]
    wos = [wo0, wo1, wo2]
    bos = [bo0, bo1, bo2]
    diffs = []
    hi = hj = None
    for i in range(3):
        d1 = jnp.maximum(
            jnp.dot(pools1[i][...], wos[i][...],
                    preferred_element_type=jnp.float32) + bos[i][...], 0.0)
        d2 = jnp.maximum(
            jnp.dot(pools2[i][...], wos[i][...],
                    preferred_element_type=jnp.float32) + bos[i][...], 0.0)
        diffs.append(jnp.exp(-(d1 - d2) ** 2))
        hi, hj = d1, d2
    diff_rep = jnp.concatenate(diffs, axis=1)
    h = jnp.maximum(
        jnp.dot(diff_rep, wc1[...], preferred_element_type=jnp.float32)
        + bc1[...], 0.0)
    score_rep = jnp.tanh(
        jnp.dot(h, wc2[...], preferred_element_type=jnp.float32) + bc2[...])
    t1 = []
    for k in range(16):
        v = jnp.dot(hi, w3[k], preferred_element_type=jnp.float32)
        t1.append(jnp.sum(v * hj, axis=1, keepdims=True))
    term_1 = jnp.concatenate(t1, axis=1)
    hij = jnp.concatenate([hi, hj], axis=1)
    term_2 = jnp.dot(hij, vt[...], preferred_element_type=jnp.float32)
    sim_rep = jnp.maximum(term_1 + term_2 + b3[...], 0.0)
    s = jnp.dot(
        jnp.maximum(jnp.dot(score_rep, ws1[...],
                            preferred_element_type=jnp.float32) + bs1[...],
                    0.0),
        ws2[...], preferred_element_type=jnp.float32) + bs2[...]
    score = jax.nn.sigmoid(s)
    t = jnp.dot(
        jnp.maximum(jnp.dot(sim_rep, wt1[...],
                            preferred_element_type=jnp.float32) + bt1[...],
                    0.0),
        wt2[...], preferred_element_type=jnp.float32) + bt2[...]
    sim_score = jax.nn.sigmoid(t)
    out_r[...] = alpha[...] * score + beta[...] * sim_score


def _head(pools1, pools2, p):
    f = 64
    args = [
        pools1[0], pools1[1], pools1[2], pools2[0], pools2[1], pools2[2],
        p['outer'][0]['W'], p['outer'][0]['b'].reshape(1, f),
        p['outer'][1]['W'], p['outer'][1]['b'].reshape(1, f),
        p['outer'][2]['W'], p['outer'][2]['b'].reshape(1, f),
        p['Wc1'], p['bc1'].reshape(1, -1),
        p['Wc2'], p['bc2'].reshape(1, -1),
        p['W3'], p['V'].T, p['b3'],
        p['Ws1'], p['bs1'].reshape(1, -1),
        p['Ws2'], p['bs2'].reshape(1, -1),
        p['Wt1'], p['bt1'].reshape(1, -1),
        p['Wt2'], p['bt2'].reshape(1, -1),
        p['alpha'].reshape(1, 1), p['beta'].reshape(1, 1),
    ]
    out = pl.pallas_call(
        _head_body,
        out_shape=jax.ShapeDtypeStruct((1, 1), jnp.float32),
    )(*args)
    return out.reshape(-1)


def kernel(x1, edge_index_1, x2, edge_index_2, params):
    p = params
    srcr = jnp.concatenate([edge_index_1[0], edge_index_2[0]])
    dstr = jnp.concatenate([edge_index_1[1], edge_index_2[1]])
    zeros = {d: jnp.zeros((N_NODES, d), jnp.float32) for d in (128, 64)}

    c1, c2 = x1, x2
    pools1, pools2 = [], []
    for i in range(3):
        agg = _segsum_both(c1, c2, srcr, dstr, zeros[c1.shape[1]])
        c1, pool1 = _dense_layer(c1, agg, 0, p['gin'][i], p['inner'][i],
                                 i == 0)
        c2, pool2 = _dense_layer(c2, agg, 1, p['gin'][i], p['inner'][i],
                                 i == 0)
        pools1.append(pool1)
        pools2.append(pool2)
    return _head(pools1, pools2, p)
